# Initial kernel scaffold; baseline (speedup 1.0000x reference)
import jax
import jax.numpy as jnp
from jax.experimental import pallas as pl


def kernel(query_states, Wc, bc, working_memory, persistent_memory, long_term_buffer):
    raise NotImplementedError("write your pallas kernel here")


# speedup vs baseline: 2.5978x; 2.5978x over previous
import functools

import jax
import jax.numpy as jnp
from jax.experimental import pallas as pl
from jax.experimental.pallas import tpu as pltpu

DIM = 1024
N_WORK = 32
N_PERSIST = 64
N_LONG = 256
M_PAD = 384
TILE = 512

_SEGMENTS = ((0, N_WORK), (N_WORK, N_WORK + N_PERSIST),
             (N_WORK + N_PERSIST, N_WORK + N_PERSIST + N_LONG))


def _body(q_ref, wc_ref, bc_ref, cmt_ref, cm_ref, o_ref):
    q = q_ref[...]
    qp = jnp.dot(q, wc_ref[...], preferred_element_type=jnp.float32)
    qp = qp + bc_ref[...]
    scale = 1.0 / jnp.sqrt(jnp.float32(DIM))
    s = jnp.dot(qp, cmt_ref[...], preferred_element_type=jnp.float32) * scale

    col = jax.lax.broadcasted_iota(jnp.int32, (1, M_PAD), 1)
    probs = jnp.zeros_like(s)
    for lo, hi in _SEGMENTS:
        mask = (col >= lo) & (col < hi)
        sm = jnp.where(mask, s, -jnp.inf)
        mx = jnp.max(sm, axis=-1, keepdims=True)
        e = jnp.where(mask, jnp.exp(s - mx), 0.0)
        denom = jnp.sum(e, axis=-1, keepdims=True)
        probs = probs + e / denom
    probs = probs * jnp.float32(1.0 / 3.0)
    o_ref[...] = jnp.dot(probs, cm_ref[...], preferred_element_type=jnp.float32)


@jax.jit
def kernel(query_states, Wc, bc, working_memory, persistent_memory,
           long_term_buffer):
    B, S, D = query_states.shape
    q2 = query_states.reshape(B * S, D)
    cmem = jnp.concatenate(
        [working_memory[0], persistent_memory[0], long_term_buffer[0],
         jnp.zeros((M_PAD - N_WORK - N_PERSIST - N_LONG, D),
                   dtype=query_states.dtype)], axis=0)
    cmt = cmem.T
    bc2 = bc.reshape(1, D)

    grid = (B * S // TILE,)
    out = pl.pallas_call(
        _body,
        grid=grid,
        in_specs=[
            pl.BlockSpec((TILE, D), lambda i: (i, 0)),
            pl.BlockSpec((D, D), lambda i: (0, 0)),
            pl.BlockSpec((1, D), lambda i: (0, 0)),
            pl.BlockSpec((D, M_PAD), lambda i: (0, 0)),
            pl.BlockSpec((M_PAD, D), lambda i: (0, 0)),
        ],
        out_specs=pl.BlockSpec((TILE, D), lambda i: (i, 0)),
        out_shape=jax.ShapeDtypeStruct((B * S, D), jnp.float32),
        compiler_params=pltpu.CompilerParams(
            dimension_semantics=("arbitrary",)),
    )(q2, Wc, bc2, cmt, cmem)
    return out.reshape(B, S, D)
